# Initial kernel scaffold; baseline (speedup 1.0000x reference)
import jax
import jax.numpy as jnp
from jax.experimental import pallas as pl


def kernel(x, token_mask, weight, e_score_correction_bias):
    raise NotImplementedError("write your pallas kernel here")


# speedup vs baseline: 1.1151x; 1.1151x over previous
import jax
import jax.numpy as jnp
from jax.experimental import pallas as pl

_N_TOK = 8192
_DIM = 2048
_N_EXPERTS = 64
_TOPK = 8
_N_GROUPS = 8
_TOPK_GROUPS = 4
_GROUP_SIZE = _N_EXPERTS // _N_GROUPS
_ROUTE_SCALE = 2.5
_BLK = 512
_NEG = -1e30


def _gate_kernel(x_ref, wt_ref, bias_ref, w_out_ref, i_out_ref):
    x = x_ref[...]
    wt = wt_ref[...]
    logits = jnp.dot(x, wt, preferred_element_type=jnp.float32)
    orig = jax.nn.sigmoid(logits)
    s = orig + bias_ref[...]

    lane = jax.lax.broadcasted_iota(jnp.int32, (_BLK, _N_EXPERTS), 1)

    gmask = (jax.lax.broadcasted_iota(jnp.int32, (_N_GROUPS, _N_EXPERTS), 1)
             // _GROUP_SIZE
             == jax.lax.broadcasted_iota(jnp.int32, (_N_GROUPS, _N_EXPERTS), 0))
    s3 = jnp.where(gmask[None, :, :], s[:, None, :], _NEG)
    m1 = jnp.max(s3, axis=2)
    lane3 = jax.lax.broadcasted_iota(
        jnp.int32, (_BLK, _N_GROUPS, _N_EXPERTS), 2)
    a1 = jnp.min(jnp.where(s3 == m1[:, :, None], lane3, _N_EXPERTS), axis=2)
    m2 = jnp.max(jnp.where(lane3 == a1[:, :, None], _NEG, s3), axis=2)
    gs = m1 + m2

    ga = gs[:, :, None]
    gb = gs[:, None, :]
    gi = jax.lax.broadcasted_iota(jnp.int32, (_BLK, _N_GROUPS, _N_GROUPS), 1)
    gj = jax.lax.broadcasted_iota(jnp.int32, (_BLK, _N_GROUPS, _N_GROUPS), 2)
    beats = (gb > ga) | ((gb == ga) & (gj < gi))
    rank = jnp.sum(beats.astype(jnp.int32), axis=2)
    keep = (rank < _TOPK_GROUPS).astype(jnp.float32)
    keep_e = jnp.max(
        jnp.where(gmask[None, :, :], keep[:, :, None], 0.0), axis=1)
    masked = s * keep_e

    work = masked
    wcols = []
    icols = []
    for _ in range(_TOPK):
        m = jnp.max(work, axis=1, keepdims=True)
        a = jnp.min(jnp.where(work == m, lane, _N_EXPERTS),
                    axis=1, keepdims=True)
        sel = lane == a
        icols.append(a)
        wcols.append(jnp.sum(jnp.where(sel, orig, 0.0), axis=1,
                             keepdims=True))
        work = jnp.where(sel, _NEG, work)
    wsel = jnp.concatenate(wcols, axis=1)
    idx = jnp.concatenate(icols, axis=1)
    wnorm = wsel / jnp.sum(wsel, axis=1, keepdims=True) * _ROUTE_SCALE

    w_out_ref[...] = wnorm
    i_out_ref[...] = idx


def kernel(x, token_mask, weight, e_score_correction_bias):
    del token_mask
    n = x.shape[0]
    wt = weight.T
    bias = e_score_correction_bias.reshape(1, _N_EXPERTS)
    grid = (n // _BLK,)
    weights, indices = pl.pallas_call(
        _gate_kernel,
        grid=grid,
        in_specs=[
            pl.BlockSpec((_BLK, _DIM), lambda i: (i, 0)),
            pl.BlockSpec((_DIM, _N_EXPERTS), lambda i: (0, 0)),
            pl.BlockSpec((1, _N_EXPERTS), lambda i: (0, 0)),
        ],
        out_specs=[
            pl.BlockSpec((_BLK, _TOPK), lambda i: (i, 0)),
            pl.BlockSpec((_BLK, _TOPK), lambda i: (i, 0)),
        ],
        out_shape=[
            jax.ShapeDtypeStruct((n, _TOPK), jnp.float32),
            jax.ShapeDtypeStruct((n, _TOPK), jnp.int32),
        ],
    )(x, wt, bias)
    return weights.astype(x.dtype), indices


# speedup vs baseline: 4.9753x; 4.4618x over previous
import jax
import jax.numpy as jnp
from jax.experimental import pallas as pl

_N_TOK = 8192
_DIM = 2048
_N_EXPERTS = 64
_TOPK = 8
_N_GROUPS = 8
_TOPK_GROUPS = 4
_GROUP_SIZE = _N_EXPERTS // _N_GROUPS
_ROUTE_SCALE = 2.5
_BLK = 512
_NEG = -1e30


def _gate_kernel(x_ref, wt_ref, bias_ref, w_out_ref, i_out_ref):
    logits = jnp.dot(x_ref[...], wt_ref[...],
                     preferred_element_type=jnp.float32)
    lt = logits.T
    orig = jax.nn.sigmoid(lt)
    s = orig + bias_ref[...]

    sg = s.reshape(_N_GROUPS, _GROUP_SIZE, _BLK)
    m1 = jnp.max(sg, axis=1)
    e_iota = jax.lax.broadcasted_iota(
        jnp.int32, (_N_GROUPS, _GROUP_SIZE, _BLK), 1)
    a1 = jnp.min(jnp.where(sg == m1[:, None, :], e_iota, _GROUP_SIZE),
                 axis=1)
    m2 = jnp.max(jnp.where(e_iota == a1[:, None, :], _NEG, sg), axis=1)
    gs = m1 + m2

    ga = gs[:, None, :]
    gb = gs[None, :, :]
    gi = jax.lax.broadcasted_iota(
        jnp.int32, (_N_GROUPS, _N_GROUPS, _BLK), 0)
    gj = jax.lax.broadcasted_iota(
        jnp.int32, (_N_GROUPS, _N_GROUPS, _BLK), 1)
    beats = (gb > ga) | ((gb == ga) & (gj < gi))
    rank = jnp.sum(beats.astype(jnp.int32), axis=1)
    keep = (rank < _TOPK_GROUPS).astype(jnp.float32)
    keep_e = jnp.broadcast_to(
        keep[:, None, :],
        (_N_GROUPS, _GROUP_SIZE, _BLK)).reshape(_N_EXPERTS, _BLK)
    masked = s * keep_e

    row = jax.lax.broadcasted_iota(jnp.int32, (_N_EXPERTS, _BLK), 0)
    work = masked
    w_rows = []
    i_rows = []
    for _ in range(_TOPK):
        m = jnp.max(work, axis=0, keepdims=True)
        a = jnp.min(jnp.where(work == m, row, _N_EXPERTS),
                    axis=0, keepdims=True)
        sel = row == a
        i_rows.append(a)
        w_rows.append(jnp.sum(jnp.where(sel, orig, 0.0), axis=0,
                              keepdims=True))
        work = jnp.where(sel, _NEG, work)
    w_t = jnp.concatenate(w_rows, axis=0)
    i_t = jnp.concatenate(i_rows, axis=0)
    w_n = w_t / jnp.sum(w_t, axis=0, keepdims=True) * _ROUTE_SCALE

    w_out_ref[...] = w_n.T
    i_out_ref[...] = i_t.T


def kernel(x, token_mask, weight, e_score_correction_bias):
    del token_mask
    n = x.shape[0]
    wt = weight.T
    bias = e_score_correction_bias.reshape(_N_EXPERTS, 1)
    grid = (n // _BLK,)
    weights, indices = pl.pallas_call(
        _gate_kernel,
        grid=grid,
        in_specs=[
            pl.BlockSpec((_BLK, _DIM), lambda i: (i, 0)),
            pl.BlockSpec((_DIM, _N_EXPERTS), lambda i: (0, 0)),
            pl.BlockSpec((_N_EXPERTS, 1), lambda i: (0, 0)),
        ],
        out_specs=[
            pl.BlockSpec((_BLK, _TOPK), lambda i: (i, 0)),
            pl.BlockSpec((_BLK, _TOPK), lambda i: (i, 0)),
        ],
        out_shape=[
            jax.ShapeDtypeStruct((n, _TOPK), jnp.float32),
            jax.ShapeDtypeStruct((n, _TOPK), jnp.int32),
        ],
    )(x, wt, bias)
    return weights.astype(x.dtype), indices


# speedup vs baseline: 5.3146x; 1.0682x over previous
import jax
import jax.numpy as jnp
from jax.experimental import pallas as pl

_N_TOK = 8192
_DIM = 2048
_N_EXPERTS = 64
_TOPK = 8
_N_GROUPS = 8
_TOPK_GROUPS = 4
_GROUP_SIZE = _N_EXPERTS // _N_GROUPS
_ROUTE_SCALE = 2.5
_BLK = 1024
_NEG = -1e30


def _gate_kernel(x_ref, wt_ref, bias_ref, w_out_ref, i_out_ref):
    logits = jnp.dot(x_ref[...], wt_ref[...],
                     preferred_element_type=jnp.float32)
    lt = logits.T
    orig = jax.nn.sigmoid(lt)
    s = orig + bias_ref[...]

    sg = s.reshape(_N_GROUPS, _GROUP_SIZE, _BLK)
    m1 = jnp.max(sg, axis=1)
    e_iota = jax.lax.broadcasted_iota(
        jnp.int32, (_N_GROUPS, _GROUP_SIZE, _BLK), 1)
    a1 = jnp.min(jnp.where(sg == m1[:, None, :], e_iota, _GROUP_SIZE),
                 axis=1)
    m2 = jnp.max(jnp.where(e_iota == a1[:, None, :], _NEG, sg), axis=1)
    gs = m1 + m2

    ga = gs[:, None, :]
    gb = gs[None, :, :]
    gi = jax.lax.broadcasted_iota(
        jnp.int32, (_N_GROUPS, _N_GROUPS, _BLK), 0)
    gj = jax.lax.broadcasted_iota(
        jnp.int32, (_N_GROUPS, _N_GROUPS, _BLK), 1)
    beats = (gb > ga) | ((gb == ga) & (gj < gi))
    rank = jnp.sum(beats.astype(jnp.int32), axis=1)
    keep = (rank < _TOPK_GROUPS).astype(jnp.float32)
    keep_e = jnp.broadcast_to(
        keep[:, None, :],
        (_N_GROUPS, _GROUP_SIZE, _BLK)).reshape(_N_EXPERTS, _BLK)
    masked = s * keep_e

    row = jax.lax.broadcasted_iota(jnp.int32, (_N_EXPERTS, _BLK), 0)
    work = masked
    w_rows = []
    i_rows = []
    for _ in range(_TOPK):
        m = jnp.max(work, axis=0, keepdims=True)
        a = jnp.min(jnp.where(work == m, row, _N_EXPERTS),
                    axis=0, keepdims=True)
        sel = row == a
        i_rows.append(a)
        w_rows.append(jnp.sum(jnp.where(sel, orig, 0.0), axis=0,
                              keepdims=True))
        work = jnp.where(sel, _NEG, work)
    w_t = jnp.concatenate(w_rows, axis=0)
    i_t = jnp.concatenate(i_rows, axis=0)
    w_n = w_t / jnp.sum(w_t, axis=0, keepdims=True) * _ROUTE_SCALE

    w_out_ref[...] = w_n.T
    i_out_ref[...] = i_t.T


def kernel(x, token_mask, weight, e_score_correction_bias):
    del token_mask
    n = x.shape[0]
    wt = weight.T
    bias = e_score_correction_bias.reshape(_N_EXPERTS, 1)
    grid = (n // _BLK,)
    weights, indices = pl.pallas_call(
        _gate_kernel,
        grid=grid,
        in_specs=[
            pl.BlockSpec((_BLK, _DIM), lambda i: (i, 0)),
            pl.BlockSpec((_DIM, _N_EXPERTS), lambda i: (0, 0)),
            pl.BlockSpec((_N_EXPERTS, 1), lambda i: (0, 0)),
        ],
        out_specs=[
            pl.BlockSpec((_BLK, _TOPK), lambda i: (i, 0)),
            pl.BlockSpec((_BLK, _TOPK), lambda i: (i, 0)),
        ],
        out_shape=[
            jax.ShapeDtypeStruct((n, _TOPK), jnp.float32),
            jax.ShapeDtypeStruct((n, _TOPK), jnp.int32),
        ],
    )(x, wt, bias)
    return weights.astype(x.dtype), indices


# speedup vs baseline: 5.5507x; 1.0444x over previous
import jax
import jax.numpy as jnp
from jax.experimental import pallas as pl

_N_TOK = 8192
_DIM = 2048
_N_EXPERTS = 64
_TOPK = 8
_N_GROUPS = 8
_TOPK_GROUPS = 4
_GROUP_SIZE = _N_EXPERTS // _N_GROUPS
_ROUTE_SCALE = 2.5
_BLK = 2048
_NEG = -1e30


def _gate_kernel(x_ref, wt_ref, bias_ref, w_out_ref, i_out_ref):
    logits = jnp.dot(x_ref[...], wt_ref[...],
                     preferred_element_type=jnp.float32)
    lt = logits.T
    orig = jax.nn.sigmoid(lt)
    s = orig + bias_ref[...]

    sg = s.reshape(_N_GROUPS, _GROUP_SIZE, _BLK)
    m1 = jnp.max(sg, axis=1)
    e_iota = jax.lax.broadcasted_iota(
        jnp.int32, (_N_GROUPS, _GROUP_SIZE, _BLK), 1)
    a1 = jnp.min(jnp.where(sg == m1[:, None, :], e_iota, _GROUP_SIZE),
                 axis=1)
    m2 = jnp.max(jnp.where(e_iota == a1[:, None, :], _NEG, sg), axis=1)
    gs = m1 + m2

    ga = gs[:, None, :]
    gb = gs[None, :, :]
    gi = jax.lax.broadcasted_iota(
        jnp.int32, (_N_GROUPS, _N_GROUPS, _BLK), 0)
    gj = jax.lax.broadcasted_iota(
        jnp.int32, (_N_GROUPS, _N_GROUPS, _BLK), 1)
    beats = (gb > ga) | ((gb == ga) & (gj < gi))
    rank = jnp.sum(beats.astype(jnp.int32), axis=1)
    keep = (rank < _TOPK_GROUPS).astype(jnp.float32)
    keep_e = jnp.broadcast_to(
        keep[:, None, :],
        (_N_GROUPS, _GROUP_SIZE, _BLK)).reshape(_N_EXPERTS, _BLK)
    masked = s * keep_e

    row = jax.lax.broadcasted_iota(jnp.int32, (_N_EXPERTS, _BLK), 0)
    work = masked
    w_rows = []
    i_rows = []
    for _ in range(_TOPK):
        m = jnp.max(work, axis=0, keepdims=True)
        a = jnp.min(jnp.where(work == m, row, _N_EXPERTS),
                    axis=0, keepdims=True)
        sel = row == a
        i_rows.append(a)
        w_rows.append(jnp.sum(jnp.where(sel, orig, 0.0), axis=0,
                              keepdims=True))
        work = jnp.where(sel, _NEG, work)
    w_t = jnp.concatenate(w_rows, axis=0)
    i_t = jnp.concatenate(i_rows, axis=0)
    w_n = w_t / jnp.sum(w_t, axis=0, keepdims=True) * _ROUTE_SCALE

    w_out_ref[...] = w_n.T
    i_out_ref[...] = i_t.T


def kernel(x, token_mask, weight, e_score_correction_bias):
    del token_mask
    n = x.shape[0]
    wt = weight.T
    bias = e_score_correction_bias.reshape(_N_EXPERTS, 1)
    grid = (n // _BLK,)
    weights, indices = pl.pallas_call(
        _gate_kernel,
        grid=grid,
        in_specs=[
            pl.BlockSpec((_BLK, _DIM), lambda i: (i, 0)),
            pl.BlockSpec((_DIM, _N_EXPERTS), lambda i: (0, 0)),
            pl.BlockSpec((_N_EXPERTS, 1), lambda i: (0, 0)),
        ],
        out_specs=[
            pl.BlockSpec((_BLK, _TOPK), lambda i: (i, 0)),
            pl.BlockSpec((_BLK, _TOPK), lambda i: (i, 0)),
        ],
        out_shape=[
            jax.ShapeDtypeStruct((n, _TOPK), jnp.float32),
            jax.ShapeDtypeStruct((n, _TOPK), jnp.int32),
        ],
    )(x, wt, bias)
    return weights.astype(x.dtype), indices
